# Initial kernel scaffold; baseline (speedup 1.0000x reference)
import jax
import jax.numpy as jnp
from jax.experimental import pallas as pl


def kernel(x, edge_index, W, att_src, att_dst, bias, gamma, beta):
    raise NotImplementedError("write your pallas kernel here")


# speedup vs baseline: 24.8303x; 24.8303x over previous
import functools

import jax
import jax.numpy as jnp
from jax import lax
from jax.experimental import pallas as pl
from jax.experimental.pallas import tpu as pltpu
from jax.experimental.pallas import tpu_sc as plsc

N_NODES = 10000
D = 128
NC = 2
NS = 16
NW = NC * NS
NP = 10240
NPS = NP // NS
CHUNK = 128


def _cdiv(a, b):
    return (a + b - 1) // b


def _tc_prep_body(x_ref, w_ref, as_ref, ad_ref, h_ref, a_ref, b_ref, u_ref):
    h = jnp.dot(x_ref[...], w_ref[...], preferred_element_type=jnp.float32)
    h_ref[...] = h
    a = jnp.sum(h * as_ref[...], axis=1)
    b = jnp.sum(h * ad_ref[...], axis=1)
    a_ref[...] = a
    b_ref[...] = b
    u_ref[...] = jnp.full((16,), jnp.max(a) + jnp.max(b), jnp.float32)


def _tc_prep(x_p, W, att_src, att_dst):
    return pl.pallas_call(
        _tc_prep_body,
        out_shape=(
            jax.ShapeDtypeStruct((NP, D), jnp.float32),
            jax.ShapeDtypeStruct((NP,), jnp.float32),
            jax.ShapeDtypeStruct((NP,), jnp.float32),
            jax.ShapeDtypeStruct((16,), jnp.float32),
        ),
    )(x_p, W, att_src, att_dst)


def _tc_finish_body(p_ref, bias_ref, gamma_ref, beta_ref, o_ref):
    o = p_ref[0] + p_ref[1] + bias_ref[...]
    o = jnp.maximum(o, 0.0)
    scale = gamma_ref[...] / jnp.sqrt(jnp.float32(1.0 + 1e-5))
    o_ref[...] = o * scale + beta_ref[...]


def _tc_finish(outp, bias, gamma, beta):
    return pl.pallas_call(
        _tc_finish_body,
        out_shape=jax.ShapeDtypeStruct((NP, D), jnp.float32),
    )(outp, bias, gamma, beta)


@functools.lru_cache(maxsize=None)
def _make_sc_alpha(T, E_TOT):
    mesh = plsc.VectorSubcoreMesh(core_axis_name="c", subcore_axis_name="s", num_cores=NC, num_subcores=NS)
    EP = NW * T

    @functools.partial(
        pl.kernel,
        out_type=(
            jax.ShapeDtypeStruct((EP,), jnp.float32),
            jax.ShapeDtypeStruct((NC, NP), jnp.float32),
        ),
        mesh=mesh,
        compiler_params=pltpu.CompilerParams(needs_layout_passes=False),
        scratch_types=[
            pltpu.VMEM((NP,), jnp.float32),
            pltpu.VMEM((NP,), jnp.float32),
            pltpu.VMEM((T,), jnp.int32),
            pltpu.VMEM((T,), jnp.int32),
            pltpu.VMEM((T,), jnp.float32),
            pltpu.VMEM((NP,), jnp.float32),
            pltpu.VMEM((16,), jnp.float32),
            pltpu.VMEM((NS, NPS), jnp.float32),
            pltpu.VMEM_SHARED((NS, NP), jnp.float32),
        ],
    )
    def sc_alpha(se_hbm, de_hbm, asrc_hbm, adst_hbm, u_hbm,
                 ex_hbm, den_hbm,
                 asrc_v, adst_v, se_v, de_v, ex_v, den_v, u_v, red_v, den_sh):
        cid = lax.axis_index("c")
        sid = lax.axis_index("s")
        wid = sid * NC + cid
        base = wid * T
        pltpu.sync_copy(asrc_hbm, asrc_v)
        pltpu.sync_copy(adst_hbm, adst_v)
        pltpu.sync_copy(se_hbm.at[pl.ds(base, T)], se_v)
        pltpu.sync_copy(de_hbm.at[pl.ds(base, T)], de_v)
        pltpu.sync_copy(u_hbm, u_v)
        uvec = plsc.load_gather(u_v, [jnp.zeros((16,), jnp.int32)])
        zeros16 = jnp.zeros((16,), jnp.float32)

        @pl.loop(0, NP // 16)
        def _zero(i):
            den_v[pl.ds(i * 16, 16)] = zeros16

        @pl.loop(0, T // 16)
        def _edges(i):
            e = i * 16
            sidx = se_v[pl.ds(e, 16)]
            didx = de_v[pl.ds(e, 16)]
            av = plsc.load_gather(asrc_v, [sidx])
            bv = plsc.load_gather(adst_v, [didx])
            al = av + bv
            al = jnp.where(al > 0, al, al * jnp.float32(0.2))
            exv = jnp.exp(al - uvec)
            eid = base + e + lax.iota(jnp.int32, 16)
            exv = jnp.where(eid < E_TOT, exv, jnp.float32(0.0))
            ex_v[pl.ds(e, 16)] = exv
            plsc.addupdate_scatter(den_v, [didx], exv)

        pltpu.sync_copy(ex_v, ex_hbm.at[pl.ds(base, T)])
        pltpu.sync_copy(den_v, den_sh.at[sid])
        plsc.subcore_barrier()
        col0 = sid * NPS
        pltpu.sync_copy(den_sh.at[:, pl.ds(col0, NPS)], red_v)

        @pl.loop(0, NPS // 16)
        def _red(ci):
            c = ci * 16
            acc = red_v[0, pl.ds(c, 16)]
            for r in range(1, NS):
                acc = acc + red_v[r, pl.ds(c, 16)]
            den_v[pl.ds(c, 16)] = acc

        pltpu.sync_copy(den_v.at[pl.ds(0, NPS)], den_hbm.at[cid, pl.ds(col0, NPS)])

    return sc_alpha


@functools.lru_cache(maxsize=None)
def _make_sc_agg(T):
    mesh = plsc.VectorSubcoreMesh(core_axis_name="c", subcore_axis_name="s", num_cores=NC, num_subcores=NS)
    NCH = T // CHUNK

    @functools.partial(
        pl.kernel,
        out_type=jax.ShapeDtypeStruct((NC, NP, D), jnp.float32),
        mesh=mesh,
        compiler_params=pltpu.CompilerParams(needs_layout_passes=False),
        scratch_types=[
            pltpu.VMEM((CHUNK,), jnp.int32),
            pltpu.VMEM((CHUNK,), jnp.int32),
            pltpu.VMEM((T,), jnp.float32),
            pltpu.VMEM((NP,), jnp.float32),
            pltpu.VMEM((2048,), jnp.float32),
            pltpu.VMEM((CHUNK, D), jnp.float32),
            pltpu.VMEM_SHARED((NP, D), jnp.float32),
            pltpu.SemaphoreType.DMA,
            pltpu.SemaphoreType.DMA,
        ],
    )
    def sc_agg(se2_hbm, de2_hbm, ex_hbm, den_hbm, h_hbm,
               out_hbm,
               se_c, de_c, coef_v, d_v, d2s, r0, acc_sh, sem0, sem1):
        cid = lax.axis_index("c")
        sid = lax.axis_index("s")
        wid = sid * NC + cid
        base = wid * T
        zeros16 = jnp.zeros((16,), jnp.float32)

        pltpu.sync_copy(den_hbm.at[0], d_v)

        @pl.loop(0, NP // 2048)
        def _dsum(b):
            pltpu.sync_copy(den_hbm.at[1, pl.ds(b * 2048, 2048)], d2s)

            @pl.loop(0, 2048 // 16)
            def _dadd(i):
                o = i * 16
                d_v[pl.ds(b * 2048 + o, 16)] = (
                    d_v[pl.ds(b * 2048 + o, 16)] + d2s[pl.ds(o, 16)])

        pltpu.sync_copy(ex_hbm.at[pl.ds(base, T)], coef_v)

        @pl.loop(0, CHUNK)
        def _zr(r):
            for c in range(D // 16):
                r0[r, pl.ds(c * 16, 16)] = zeros16

        @pl.loop(0, NPS // CHUNK)
        def _zacc(b):
            pltpu.sync_copy(r0, acc_sh.at[pl.ds(sid * NPS + b * CHUNK, CHUNK)])

        plsc.subcore_barrier()

        @pl.loop(0, NCH)
        def _chunk(j):
            pltpu.async_copy(se2_hbm.at[wid, j], se_c, sem0)
            pltpu.async_copy(de2_hbm.at[wid, j], de_c, sem1).wait()
            pltpu.make_async_copy(se2_hbm.at[wid, j], se_c, sem0).wait()

            @pl.loop(0, CHUNK // 16)
            def _coef(i):
                e = i * 16
                didx = de_c[pl.ds(e, 16)]
                dv = plsc.load_gather(d_v, [didx])
                g = j * CHUNK + e
                coef_v[pl.ds(g, 16)] = (
                    coef_v[pl.ds(g, 16)] / (dv + jnp.float32(1e-16)))

            pltpu.async_copy(h_hbm.at[se_c], r0, sem0).wait()

            @pl.loop(0, CHUNK)
            def _scale(r):
                cj = plsc.load_gather(
                    coef_v, [jnp.full((16,), j * CHUNK + r, jnp.int32)])
                for c in range(D // 16):
                    r0[r, pl.ds(c * 16, 16)] = r0[r, pl.ds(c * 16, 16)] * cj

            pltpu.async_copy(r0, acc_sh.at[de_c], sem1, add=True).wait()

        plsc.subcore_barrier()
        row0 = sid * NPS
        pltpu.sync_copy(acc_sh.at[pl.ds(row0, NPS)],
                        out_hbm.at[cid, pl.ds(row0, NPS)])

    return sc_agg


def kernel(x, edge_index, W, att_src, att_dst, bias, gamma, beta):
    N = x.shape[0]
    E = edge_index.shape[1]
    E_TOT = E + N
    NCH = _cdiv(E_TOT, NW * CHUNK)
    T = NCH * CHUNK
    EP = NW * T
    PAD = EP - E_TOT

    loops = jnp.arange(N, dtype=jnp.int32)
    src = jnp.concatenate([
        edge_index[0].astype(jnp.int32), loops,
        jnp.zeros((PAD,), jnp.int32)])
    dst = jnp.concatenate([
        edge_index[1].astype(jnp.int32), loops,
        jnp.full((PAD,), N, jnp.int32)])

    x_p = jnp.pad(x, ((0, NP - N), (0, 0)))
    h, a_src_n, a_dst_n, u = _tc_prep(
        x_p, W, att_src.reshape(1, D), att_dst.reshape(1, D))

    ex, den = _make_sc_alpha(T, E_TOT)(src, dst, a_src_n, a_dst_n, u)

    outp = _make_sc_agg(T)(
        src.reshape(NW, -1, CHUNK), dst.reshape(NW, -1, CHUNK), ex, den, h)

    out_full = _tc_finish(outp, bias.reshape(1, D), gamma.reshape(1, D),
                          beta.reshape(1, D))
    return out_full[:N]


# speedup vs baseline: 36.0983x; 1.4538x over previous
import functools

import jax
import jax.numpy as jnp
from jax import lax
from jax.experimental import pallas as pl
from jax.experimental.pallas import tpu as pltpu
from jax.experimental.pallas import tpu_sc as plsc

N_NODES = 10000
D = 128
NC = 2
NS = 16
NW = NC * NS
NP = 10240
NPS = NP // NS
CHUNK = 128


def _cdiv(a, b):
    return (a + b - 1) // b


def _tc_prep_body(x_ref, w_ref, as_ref, ad_ref, h_ref, a_ref, b_ref, u_ref):
    h = jnp.dot(x_ref[...], w_ref[...], preferred_element_type=jnp.float32)
    h_ref[...] = h
    a = jnp.sum(h * as_ref[...], axis=1)
    b = jnp.sum(h * ad_ref[...], axis=1)
    a_ref[...] = a
    b_ref[...] = b
    u_ref[...] = jnp.full((16,), jnp.max(a) + jnp.max(b), jnp.float32)


def _tc_prep(x_p, W, att_src, att_dst):
    return pl.pallas_call(
        _tc_prep_body,
        out_shape=(
            jax.ShapeDtypeStruct((NP, D), jnp.float32),
            jax.ShapeDtypeStruct((NP,), jnp.float32),
            jax.ShapeDtypeStruct((NP,), jnp.float32),
            jax.ShapeDtypeStruct((16,), jnp.float32),
        ),
    )(x_p, W, att_src, att_dst)


def _tc_finish_body(p_ref, bias_ref, gamma_ref, beta_ref, o_ref):
    o = p_ref[0] + p_ref[1] + bias_ref[...]
    o = jnp.maximum(o, 0.0)
    scale = gamma_ref[...] / jnp.sqrt(jnp.float32(1.0 + 1e-5))
    o_ref[...] = o * scale + beta_ref[...]


def _tc_finish(outp, bias, gamma, beta):
    return pl.pallas_call(
        _tc_finish_body,
        out_shape=jax.ShapeDtypeStruct((NP, D), jnp.float32),
    )(outp, bias, gamma, beta)


@functools.lru_cache(maxsize=None)
def _make_sc_alpha(T, E_TOT):
    mesh = plsc.VectorSubcoreMesh(core_axis_name="c", subcore_axis_name="s", num_cores=NC, num_subcores=NS)
    EP = NW * T

    @functools.partial(
        pl.kernel,
        out_type=(
            jax.ShapeDtypeStruct((EP,), jnp.float32),
            jax.ShapeDtypeStruct((NC, NP), jnp.float32),
        ),
        mesh=mesh,
        compiler_params=pltpu.CompilerParams(needs_layout_passes=False),
        scratch_types=[
            pltpu.VMEM((NP,), jnp.float32),
            pltpu.VMEM((NP,), jnp.float32),
            pltpu.VMEM((T,), jnp.int32),
            pltpu.VMEM((T,), jnp.int32),
            pltpu.VMEM((T,), jnp.float32),
            pltpu.VMEM((NP,), jnp.float32),
            pltpu.VMEM((16,), jnp.float32),
            pltpu.VMEM((NS, NPS), jnp.float32),
            pltpu.VMEM_SHARED((NS, NP), jnp.float32),
        ],
    )
    def sc_alpha(se_hbm, de_hbm, asrc_hbm, adst_hbm, u_hbm,
                 ex_hbm, den_hbm,
                 asrc_v, adst_v, se_v, de_v, ex_v, den_v, u_v, red_v, den_sh):
        cid = lax.axis_index("c")
        sid = lax.axis_index("s")
        wid = sid * NC + cid
        base = wid * T
        pltpu.sync_copy(asrc_hbm, asrc_v)
        pltpu.sync_copy(adst_hbm, adst_v)
        pltpu.sync_copy(se_hbm.at[pl.ds(base, T)], se_v)
        pltpu.sync_copy(de_hbm.at[pl.ds(base, T)], de_v)
        pltpu.sync_copy(u_hbm, u_v)
        uvec = plsc.load_gather(u_v, [jnp.zeros((16,), jnp.int32)])
        zeros16 = jnp.zeros((16,), jnp.float32)

        @pl.loop(0, NP // 16)
        def _zero(i):
            den_v[pl.ds(i * 16, 16)] = zeros16

        @pl.loop(0, T // 16)
        def _edges(i):
            e = i * 16
            sidx = se_v[pl.ds(e, 16)]
            didx = de_v[pl.ds(e, 16)]
            av = plsc.load_gather(asrc_v, [sidx])
            bv = plsc.load_gather(adst_v, [didx])
            al = av + bv
            al = jnp.where(al > 0, al, al * jnp.float32(0.2))
            exv = jnp.exp(al - uvec)
            eid = base + e + lax.iota(jnp.int32, 16)
            exv = jnp.where(eid < E_TOT, exv, jnp.float32(0.0))
            ex_v[pl.ds(e, 16)] = exv
            plsc.addupdate_scatter(den_v, [didx], exv)

        pltpu.sync_copy(ex_v, ex_hbm.at[pl.ds(base, T)])
        pltpu.sync_copy(den_v, den_sh.at[sid])
        plsc.subcore_barrier()
        col0 = sid * NPS
        pltpu.sync_copy(den_sh.at[:, pl.ds(col0, NPS)], red_v)

        @pl.loop(0, NPS // 16)
        def _red(ci):
            c = ci * 16
            acc = red_v[0, pl.ds(c, 16)]
            for r in range(1, NS):
                acc = acc + red_v[r, pl.ds(c, 16)]
            den_v[pl.ds(c, 16)] = acc

        pltpu.sync_copy(den_v.at[pl.ds(0, NPS)], den_hbm.at[cid, pl.ds(col0, NPS)])

    return sc_alpha


@functools.lru_cache(maxsize=None)
def _make_sc_agg(T):
    mesh = plsc.VectorSubcoreMesh(core_axis_name="c", subcore_axis_name="s", num_cores=NC, num_subcores=NS)
    NCH = T // CHUNK

    @functools.partial(
        pl.kernel,
        out_type=jax.ShapeDtypeStruct((NC, NP, D), jnp.float32),
        mesh=mesh,
        compiler_params=pltpu.CompilerParams(needs_layout_passes=False),
        scratch_types=[
            pltpu.VMEM((CHUNK,), jnp.int32),
            pltpu.VMEM((CHUNK,), jnp.int32),
            pltpu.VMEM((CHUNK,), jnp.int32),
            pltpu.VMEM((CHUNK,), jnp.int32),
            pltpu.VMEM((CHUNK,), jnp.float32),
            pltpu.VMEM((CHUNK,), jnp.float32),
            pltpu.VMEM((CHUNK, D), jnp.float32),
            pltpu.VMEM((CHUNK, D), jnp.float32),
            pltpu.VMEM((NP,), jnp.float32),
            pltpu.VMEM((2048,), jnp.float32),
            pltpu.VMEM_SHARED((NP, D), jnp.float32),
            pltpu.SemaphoreType.DMA,
            pltpu.SemaphoreType.DMA,
            pltpu.SemaphoreType.DMA,
            pltpu.SemaphoreType.DMA,
            pltpu.SemaphoreType.DMA,
            pltpu.SemaphoreType.DMA,
            pltpu.SemaphoreType.DMA,
            pltpu.SemaphoreType.DMA,
            pltpu.SemaphoreType.DMA,
            pltpu.SemaphoreType.DMA,
        ],
    )
    def sc_agg(se2_hbm, de2_hbm, ex2_hbm, den_hbm, h_hbm,
               out_hbm,
               se0, se1, de0, de1, ex0, ex1, r0, r1, d_v, d2s, acc_sh,
               sem_se0, sem_se1, sem_de0, sem_de1, sem_ex0, sem_ex1,
               sem_g0, sem_g1, sem_s0, sem_s1):
        cid = lax.axis_index("c")
        sid = lax.axis_index("s")
        wid = sid * NC + cid
        zeros16 = jnp.zeros((16,), jnp.float32)
        bufs = [
            (se0, de0, ex0, r0, sem_se0, sem_de0, sem_ex0, sem_g0, sem_s0),
            (se1, de1, ex1, r1, sem_se1, sem_de1, sem_ex1, sem_g1, sem_s1),
        ]

        pltpu.sync_copy(den_hbm.at[0], d_v)

        @pl.loop(0, NP // 2048)
        def _dsum(b):
            pltpu.sync_copy(den_hbm.at[1, pl.ds(b * 2048, 2048)], d2s)

            @pl.loop(0, 2048 // 16)
            def _dadd(i):
                o = i * 16
                d_v[pl.ds(b * 2048 + o, 16)] = (
                    d_v[pl.ds(b * 2048 + o, 16)] + d2s[pl.ds(o, 16)])

        @pl.loop(0, CHUNK)
        def _zr(r):
            for c in range(D // 16):
                r0[r, pl.ds(c * 16, 16)] = zeros16

        @pl.loop(0, NPS // CHUNK)
        def _zacc(b):
            pltpu.sync_copy(r0, acc_sh.at[pl.ds(sid * NPS + b * CHUNK, CHUNK)])

        plsc.subcore_barrier()

        def body(j, cur, nxt):
            cse, cde, cex, cr, csem_se, csem_de, csem_ex, csem_g, csem_s = cur
            nse, nde, nex, nr, nsem_se, nsem_de, nsem_ex, nsem_g, nsem_s = nxt

            pltpu.make_async_copy(de2_hbm.at[wid, j], cde, csem_de).wait()
            pltpu.make_async_copy(ex2_hbm.at[wid, j], cex, csem_ex).wait()

            @pl.loop(0, CHUNK // 16)
            def _coef(i):
                e = i * 16
                didx = cde[pl.ds(e, 16)]
                dv = plsc.load_gather(d_v, [didx])
                cex[pl.ds(e, 16)] = (
                    cex[pl.ds(e, 16)] / (dv + jnp.float32(1e-16)))

            pltpu.make_async_copy(h_hbm.at[cse], cr, csem_g).wait()

            @pl.when(j > 0)
            def _():
                pltpu.make_async_copy(nr, acc_sh.at[nde], nsem_s).wait()

            @pl.when(j + 1 < NCH)
            def _():
                pltpu.make_async_copy(
                    se2_hbm.at[wid, j + 1], nse, nsem_se).wait()
                pltpu.async_copy(h_hbm.at[nse], nr, nsem_g)
                pltpu.async_copy(de2_hbm.at[wid, j + 1], nde, nsem_de)
                pltpu.async_copy(ex2_hbm.at[wid, j + 1], nex, nsem_ex)

            @pl.loop(0, CHUNK, unroll=2)
            def _scale(r):
                cj = plsc.load_gather(cex, [jnp.full((16,), r, jnp.int32)])
                for c in range(D // 16):
                    cr[r, pl.ds(c * 16, 16)] = cr[r, pl.ds(c * 16, 16)] * cj

            pltpu.async_copy(cr, acc_sh.at[cde], csem_s, add=True)

            @pl.when(j + 2 < NCH)
            def _():
                pltpu.async_copy(se2_hbm.at[wid, j + 2], cse, csem_se)

        pltpu.async_copy(se2_hbm.at[wid, 0], se0, sem_se0)
        pltpu.async_copy(de2_hbm.at[wid, 0], de0, sem_de0)
        pltpu.async_copy(ex2_hbm.at[wid, 0], ex0, sem_ex0)
        pltpu.async_copy(se2_hbm.at[wid, 1], se1, sem_se1)
        pltpu.make_async_copy(se2_hbm.at[wid, 0], se0, sem_se0).wait()
        pltpu.async_copy(h_hbm.at[se0], r0, sem_g0)

        @pl.loop(0, NCH)
        def _chunk(j):
            @pl.when(j % 2 == 0)
            def _():
                body(j, bufs[0], bufs[1])

            @pl.when(j % 2 == 1)
            def _():
                body(j, bufs[1], bufs[0])

        lse, lde, lex, lr, _, _, _, _, lsem_s = bufs[(NCH - 1) % 2]
        pltpu.make_async_copy(lr, acc_sh.at[lde], lsem_s).wait()

        plsc.subcore_barrier()
        row0 = sid * NPS
        pltpu.sync_copy(acc_sh.at[pl.ds(row0, NPS)],
                        out_hbm.at[cid, pl.ds(row0, NPS)])

    return sc_agg


def kernel(x, edge_index, W, att_src, att_dst, bias, gamma, beta):
    N = x.shape[0]
    E = edge_index.shape[1]
    E_TOT = E + N
    NCH = _cdiv(E_TOT, NW * CHUNK)
    T = NCH * CHUNK
    EP = NW * T
    PAD = EP - E_TOT

    loops = jnp.arange(N, dtype=jnp.int32)
    src = jnp.concatenate([
        edge_index[0].astype(jnp.int32), loops,
        jnp.zeros((PAD,), jnp.int32)])
    dst = jnp.concatenate([
        edge_index[1].astype(jnp.int32), loops,
        jnp.full((PAD,), N, jnp.int32)])

    x_p = jnp.pad(x, ((0, NP - N), (0, 0)))
    h, a_src_n, a_dst_n, u = _tc_prep(
        x_p, W, att_src.reshape(1, D), att_dst.reshape(1, D))

    ex, den = _make_sc_alpha(T, E_TOT)(src, dst, a_src_n, a_dst_n, u)

    outp = _make_sc_agg(T)(
        src.reshape(NW, -1, CHUNK), dst.reshape(NW, -1, CHUNK),
        ex.reshape(NW, -1, CHUNK), den, h)

    out_full = _tc_finish(outp, bias.reshape(1, D), gamma.reshape(1, D),
                          beta.reshape(1, D))
    return out_full[:N]


# speedup vs baseline: 36.1828x; 1.0023x over previous
import functools

import jax
import jax.numpy as jnp
from jax import lax
from jax.experimental import pallas as pl
from jax.experimental.pallas import tpu as pltpu
from jax.experimental.pallas import tpu_sc as plsc

N_NODES = 10000
D = 128
NC = 2
NS = 16
NW = NC * NS
NP = 10240
NPS = NP // NS
CHUNK = 128


def _cdiv(a, b):
    return (a + b - 1) // b


def _tc_prep_body(x_ref, w_ref, as_ref, ad_ref, h_ref, a_ref, b_ref, u_ref):
    h = jnp.dot(x_ref[...], w_ref[...], preferred_element_type=jnp.float32)
    h_ref[...] = h
    a = jnp.sum(h * as_ref[...], axis=1)
    b = jnp.sum(h * ad_ref[...], axis=1)
    a_ref[...] = a
    b_ref[...] = b
    u_ref[...] = jnp.full((16,), jnp.max(a) + jnp.max(b), jnp.float32)


def _tc_prep(x_p, W, att_src, att_dst):
    return pl.pallas_call(
        _tc_prep_body,
        out_shape=(
            jax.ShapeDtypeStruct((NP, D), jnp.float32),
            jax.ShapeDtypeStruct((NP,), jnp.float32),
            jax.ShapeDtypeStruct((NP,), jnp.float32),
            jax.ShapeDtypeStruct((16,), jnp.float32),
        ),
    )(x_p, W, att_src, att_dst)


def _tc_finish_body(p_ref, bias_ref, gamma_ref, beta_ref, o_ref):
    o = p_ref[0] + p_ref[1] + bias_ref[...]
    o = jnp.maximum(o, 0.0)
    scale = gamma_ref[...] / jnp.sqrt(jnp.float32(1.0 + 1e-5))
    o_ref[...] = o * scale + beta_ref[...]


def _tc_finish(outp, bias, gamma, beta):
    return pl.pallas_call(
        _tc_finish_body,
        out_shape=jax.ShapeDtypeStruct((NP, D), jnp.float32),
    )(outp, bias, gamma, beta)


@functools.lru_cache(maxsize=None)
def _make_sc_alpha(T, E_TOT):
    mesh = plsc.VectorSubcoreMesh(core_axis_name="c", subcore_axis_name="s", num_cores=NC, num_subcores=NS)
    EP = NW * T

    @functools.partial(
        pl.kernel,
        out_type=(
            jax.ShapeDtypeStruct((EP,), jnp.float32),
            jax.ShapeDtypeStruct((NC, NP), jnp.float32),
        ),
        mesh=mesh,
        compiler_params=pltpu.CompilerParams(needs_layout_passes=False),
        scratch_types=[
            pltpu.VMEM((NP,), jnp.float32),
            pltpu.VMEM((NP,), jnp.float32),
            pltpu.VMEM((T,), jnp.int32),
            pltpu.VMEM((T,), jnp.int32),
            pltpu.VMEM((T,), jnp.float32),
            pltpu.VMEM((NP,), jnp.float32),
            pltpu.VMEM((16,), jnp.float32),
            pltpu.VMEM((NS, NPS), jnp.float32),
            pltpu.VMEM_SHARED((NS, NP), jnp.float32),
        ],
    )
    def sc_alpha(se_hbm, de_hbm, asrc_hbm, adst_hbm, u_hbm,
                 ex_hbm, den_hbm,
                 asrc_v, adst_v, se_v, de_v, ex_v, den_v, u_v, red_v, den_sh):
        cid = lax.axis_index("c")
        sid = lax.axis_index("s")
        wid = sid * NC + cid
        base = wid * T
        pltpu.sync_copy(asrc_hbm, asrc_v)
        pltpu.sync_copy(adst_hbm, adst_v)
        pltpu.sync_copy(se_hbm.at[pl.ds(base, T)], se_v)
        pltpu.sync_copy(de_hbm.at[pl.ds(base, T)], de_v)
        pltpu.sync_copy(u_hbm, u_v)
        uvec = plsc.load_gather(u_v, [jnp.zeros((16,), jnp.int32)])
        zeros16 = jnp.zeros((16,), jnp.float32)

        @pl.loop(0, NP // 16)
        def _zero(i):
            den_v[pl.ds(i * 16, 16)] = zeros16

        @pl.loop(0, T // 16)
        def _edges(i):
            e = i * 16
            sidx = se_v[pl.ds(e, 16)]
            didx = de_v[pl.ds(e, 16)]
            av = plsc.load_gather(asrc_v, [sidx])
            bv = plsc.load_gather(adst_v, [didx])
            al = av + bv
            al = jnp.where(al > 0, al, al * jnp.float32(0.2))
            exv = jnp.exp(al - uvec)
            eid = base + e + lax.iota(jnp.int32, 16)
            exv = jnp.where(eid < E_TOT, exv, jnp.float32(0.0))
            ex_v[pl.ds(e, 16)] = exv
            plsc.addupdate_scatter(den_v, [didx], exv)

        pltpu.sync_copy(ex_v, ex_hbm.at[pl.ds(base, T)])
        pltpu.sync_copy(den_v, den_sh.at[sid])
        plsc.subcore_barrier()
        col0 = sid * NPS
        pltpu.sync_copy(den_sh.at[:, pl.ds(col0, NPS)], red_v)

        @pl.loop(0, NPS // 16)
        def _red(ci):
            c = ci * 16
            acc = red_v[0, pl.ds(c, 16)]
            for r in range(1, NS):
                acc = acc + red_v[r, pl.ds(c, 16)]
            den_v[pl.ds(c, 16)] = acc

        pltpu.sync_copy(den_v.at[pl.ds(0, NPS)], den_hbm.at[cid, pl.ds(col0, NPS)])

    return sc_alpha


@functools.lru_cache(maxsize=None)
def _make_sc_agg(T):
    mesh = plsc.VectorSubcoreMesh(core_axis_name="c", subcore_axis_name="s", num_cores=NC, num_subcores=NS)
    NCH = T // CHUNK

    @functools.partial(
        pl.kernel,
        out_type=jax.ShapeDtypeStruct((NC, NP, D), jnp.float32),
        mesh=mesh,
        compiler_params=pltpu.CompilerParams(needs_layout_passes=False),
        scratch_types=[
            pltpu.VMEM((CHUNK,), jnp.int32),
            pltpu.VMEM((CHUNK,), jnp.int32),
            pltpu.VMEM((CHUNK,), jnp.int32),
            pltpu.VMEM((CHUNK,), jnp.int32),
            pltpu.VMEM((CHUNK,), jnp.float32),
            pltpu.VMEM((CHUNK,), jnp.float32),
            pltpu.VMEM((CHUNK, D), jnp.float32),
            pltpu.VMEM((CHUNK, D), jnp.float32),
            pltpu.VMEM((NP,), jnp.float32),
            pltpu.VMEM((2048,), jnp.float32),
            pltpu.VMEM_SHARED((NP, D), jnp.float32),
            pltpu.SemaphoreType.DMA,
            pltpu.SemaphoreType.DMA,
            pltpu.SemaphoreType.DMA,
            pltpu.SemaphoreType.DMA,
            pltpu.SemaphoreType.DMA,
            pltpu.SemaphoreType.DMA,
            pltpu.SemaphoreType.DMA,
            pltpu.SemaphoreType.DMA,
            pltpu.SemaphoreType.DMA,
            pltpu.SemaphoreType.DMA,
        ],
    )
    def sc_agg(se2_hbm, de2_hbm, ex2_hbm, den_hbm, h_hbm,
               out_hbm,
               se0, se1, de0, de1, ex0, ex1, r0, r1, d_v, d2s, acc_sh,
               sem_se0, sem_se1, sem_de0, sem_de1, sem_ex0, sem_ex1,
               sem_g0, sem_g1, sem_s0, sem_s1):
        cid = lax.axis_index("c")
        sid = lax.axis_index("s")
        wid = sid * NC + cid
        zeros16 = jnp.zeros((16,), jnp.float32)
        bufs = [
            (se0, de0, ex0, r0, sem_se0, sem_de0, sem_ex0, sem_g0, sem_s0),
            (se1, de1, ex1, r1, sem_se1, sem_de1, sem_ex1, sem_g1, sem_s1),
        ]

        pltpu.sync_copy(den_hbm.at[0], d_v)

        @pl.loop(0, NP // 2048)
        def _dsum(b):
            pltpu.sync_copy(den_hbm.at[1, pl.ds(b * 2048, 2048)], d2s)

            @pl.loop(0, 2048 // 16)
            def _dadd(i):
                o = i * 16
                d_v[pl.ds(b * 2048 + o, 16)] = (
                    d_v[pl.ds(b * 2048 + o, 16)] + d2s[pl.ds(o, 16)])

        @pl.loop(0, CHUNK)
        def _zr(r):
            for c in range(D // 16):
                r0[r, pl.ds(c * 16, 16)] = zeros16

        @pl.loop(0, NPS // CHUNK)
        def _zacc(b):
            pltpu.sync_copy(r0, acc_sh.at[pl.ds(sid * NPS + b * CHUNK, CHUNK)])

        plsc.subcore_barrier()

        def body(j, cur, nxt):
            cse, cde, cex, cr, csem_se, csem_de, csem_ex, csem_g, csem_s = cur
            nse, nde, nex, nr, nsem_se, nsem_de, nsem_ex, nsem_g, nsem_s = nxt

            pltpu.make_async_copy(de2_hbm.at[wid, j], cde, csem_de).wait()
            pltpu.make_async_copy(ex2_hbm.at[wid, j], cex, csem_ex).wait()

            @pl.loop(0, CHUNK // 16, unroll=2)
            def _coef(i):
                e = i * 16
                didx = cde[pl.ds(e, 16)]
                dv = plsc.load_gather(d_v, [didx])
                cex[pl.ds(e, 16)] = (
                    cex[pl.ds(e, 16)] / (dv + jnp.float32(1e-16)))

            pltpu.make_async_copy(h_hbm.at[cse], cr, csem_g).wait()

            @pl.when(j > 0)
            def _():
                pltpu.make_async_copy(nr, acc_sh.at[nde], nsem_s).wait()

            @pl.when(j + 1 < NCH)
            def _():
                pltpu.make_async_copy(
                    se2_hbm.at[wid, j + 1], nse, nsem_se).wait()
                pltpu.async_copy(h_hbm.at[nse], nr, nsem_g)
                pltpu.async_copy(de2_hbm.at[wid, j + 1], nde, nsem_de)
                pltpu.async_copy(ex2_hbm.at[wid, j + 1], nex, nsem_ex)

            @pl.loop(0, CHUNK, unroll=4)
            def _scale(r):
                cj = plsc.load_gather(cex, [jnp.full((16,), r, jnp.int32)])
                for c in range(D // 16):
                    cr[r, pl.ds(c * 16, 16)] = cr[r, pl.ds(c * 16, 16)] * cj

            pltpu.async_copy(cr, acc_sh.at[cde], csem_s, add=True)

            @pl.when(j + 2 < NCH)
            def _():
                pltpu.async_copy(se2_hbm.at[wid, j + 2], cse, csem_se)

        pltpu.async_copy(se2_hbm.at[wid, 0], se0, sem_se0)
        pltpu.async_copy(de2_hbm.at[wid, 0], de0, sem_de0)
        pltpu.async_copy(ex2_hbm.at[wid, 0], ex0, sem_ex0)
        pltpu.async_copy(se2_hbm.at[wid, 1], se1, sem_se1)
        pltpu.make_async_copy(se2_hbm.at[wid, 0], se0, sem_se0).wait()
        pltpu.async_copy(h_hbm.at[se0], r0, sem_g0)

        @pl.loop(0, NCH)
        def _chunk(j):
            @pl.when(j % 2 == 0)
            def _():
                body(j, bufs[0], bufs[1])

            @pl.when(j % 2 == 1)
            def _():
                body(j, bufs[1], bufs[0])

        lse, lde, lex, lr, _, _, _, _, lsem_s = bufs[(NCH - 1) % 2]
        pltpu.make_async_copy(lr, acc_sh.at[lde], lsem_s).wait()

        plsc.subcore_barrier()
        row0 = sid * NPS
        pltpu.sync_copy(acc_sh.at[pl.ds(row0, NPS)],
                        out_hbm.at[cid, pl.ds(row0, NPS)])

    return sc_agg


def kernel(x, edge_index, W, att_src, att_dst, bias, gamma, beta):
    N = x.shape[0]
    E = edge_index.shape[1]
    E_TOT = E + N
    NCH = _cdiv(E_TOT, NW * CHUNK)
    T = NCH * CHUNK
    EP = NW * T
    PAD = EP - E_TOT

    loops = jnp.arange(N, dtype=jnp.int32)
    src = jnp.concatenate([
        edge_index[0].astype(jnp.int32), loops,
        jnp.zeros((PAD,), jnp.int32)])
    dst = jnp.concatenate([
        edge_index[1].astype(jnp.int32), loops,
        jnp.full((PAD,), N, jnp.int32)])

    x_p = jnp.pad(x, ((0, NP - N), (0, 0)))
    h, a_src_n, a_dst_n, u = _tc_prep(
        x_p, W, att_src.reshape(1, D), att_dst.reshape(1, D))

    ex, den = _make_sc_alpha(T, E_TOT)(src, dst, a_src_n, a_dst_n, u)

    outp = _make_sc_agg(T)(
        src.reshape(NW, -1, CHUNK), dst.reshape(NW, -1, CHUNK),
        ex.reshape(NW, -1, CHUNK), den, h)

    out_full = _tc_finish(outp, bias.reshape(1, D), gamma.reshape(1, D),
                          beta.reshape(1, D))
    return out_full[:N]
